# baseline (speedup 1.0000x reference)
import jax
import jax.numpy as jnp
from jax.experimental import pallas as pl
from jax.experimental.pallas import tpu as pltpu

B = 1024
K = 40960
KBLK = 2048
NK = K // KBLK


def _body(wfts_ref, bfts_ref, stm_ref, Wft_ref, bft_ref, W1_ref, b1_ref,
          W2_ref, b2_ref, out_ref, accw_ref, accb_ref):
    k = pl.program_id(0)

    @pl.when(k == 0)
    def _():
        accw_ref[...] = jnp.zeros_like(accw_ref)
        accb_ref[...] = jnp.zeros_like(accb_ref)

    dn = (((1,), (1,)), ((), ()))
    accw_ref[...] += jax.lax.dot_general(
        wfts_ref[...], Wft_ref[...], dn, preferred_element_type=jnp.float32)
    accb_ref[...] += jax.lax.dot_general(
        bfts_ref[...], Wft_ref[...], dn, preferred_element_type=jnp.float32)

    @pl.when(k == NK - 1)
    def _():
        w = accw_ref[...] + bft_ref[...]
        b = accb_ref[...] + bft_ref[...]
        stm = stm_ref[...]
        cat_wb = jnp.concatenate([w, b], axis=1)
        cat_bw = jnp.concatenate([b, w], axis=1)
        acc = stm * cat_wb + (1.0 - stm) * cat_bw
        x1 = jnp.clip(acc, 0.0, 1.0)
        h = jax.lax.dot_general(x1, W1_ref[...], dn,
                                preferred_element_type=jnp.float32)
        h = jnp.clip(h + b1_ref[...], 0.0, 1.0)
        out = jax.lax.dot_general(h, W2_ref[...], dn,
                                  preferred_element_type=jnp.float32)
        out_ref[...] = out + b2_ref[0]


def kernel(wfts, bfts, stm, W_ft, b_ft, W1, b1, W2, b2):
    grid = (NK,)
    out = pl.pallas_call(
        _body,
        grid=grid,
        in_specs=[
            pl.BlockSpec((B, KBLK), lambda k: (0, k)),
            pl.BlockSpec((B, KBLK), lambda k: (0, k)),
            pl.BlockSpec((B, 1), lambda k: (0, 0)),
            pl.BlockSpec((4, KBLK), lambda k: (0, k)),
            pl.BlockSpec((1, 4), lambda k: (0, 0)),
            pl.BlockSpec((8, 8), lambda k: (0, 0)),
            pl.BlockSpec((1, 8), lambda k: (0, 0)),
            pl.BlockSpec((8, 8), lambda k: (0, 0)),
            pl.BlockSpec(memory_space=pltpu.SMEM),
        ],
        out_specs=pl.BlockSpec((B, 8), lambda k: (0, 0)),
        out_shape=jax.ShapeDtypeStruct((B, 8), jnp.float32),
        scratch_shapes=[
            pltpu.VMEM((B, 4), jnp.float32),
            pltpu.VMEM((B, 4), jnp.float32),
        ],
    )(wfts, bfts, stm, W_ft,
      b_ft.reshape(1, 4), W1, b1.reshape(1, 8),
      jnp.zeros((8, 8), jnp.float32).at[0, :].set(W2[0]), b2)
    return out[:, 0:1]
